# baseline (speedup 1.0000x reference)
import functools

import jax
import jax.numpy as jnp
from jax import lax
from jax.experimental import pallas as pl
from jax.experimental.pallas import tpu as pltpu
from jax.experimental.pallas import tpu_sc as plsc

EMB = 64
HID = 128
B = 1024
T = 50
TA = 40
TB = T - TA
CHUNK = 80
LANES = 16


def _make_gather(ntok):
    info = plsc.get_sparse_core_info()
    nc, ns = info.num_cores, info.num_subcores
    nw = nc * ns
    tok_w = ntok // nw
    nchunk = tok_w // CHUNK

    mesh = plsc.VectorSubcoreMesh(core_axis_name="c", subcore_axis_name="s")

    @functools.partial(
        pl.kernel,
        mesh=mesh,
        compiler_params=pltpu.CompilerParams(needs_layout_passes=False),
        out_type=jax.ShapeDtypeStruct((ntok, EMB), jnp.float32),
        scratch_types=[
            pltpu.VMEM((tok_w,), jnp.int32),
            pltpu.VMEM((CHUNK, EMB), jnp.float32),
            pltpu.VMEM((CHUNK, EMB), jnp.float32),
            pltpu.SemaphoreType.DMA,
            pltpu.SemaphoreType.DMA,
        ],
    )
    def gather_k(table_hbm, idx_hbm, out_hbm, idx_v, rows0, rows1,
                 sem0, sem1):
        wid = lax.axis_index("s") * nc + lax.axis_index("c")
        base = wid * tok_w
        pltpu.sync_copy(idx_hbm.at[wid], idx_v)
        bufs = ((rows0, sem0), (rows1, sem1))

        def fire(g, rows_v, sem):
            def fq(q, c):
                iv = idx_v[pl.ds(g * CHUNK + q * LANES, LANES)]
                for jj in range(LANES):
                    pltpu.async_copy(table_hbm.at[iv[jj]],
                                     rows_v.at[q * LANES + jj], sem)
                return c
            lax.fori_loop(0, CHUNK // LANES, fq, 0)

        def drain_out(g, rows_v, sem):
            def dj(j, c):
                pltpu.make_async_copy(table_hbm.at[0], rows_v.at[j],
                                      sem).wait()
                return c
            lax.fori_loop(0, CHUNK, dj, 0)
            pltpu.sync_copy(rows_v,
                            out_hbm.at[pl.ds(base + g * CHUNK, CHUNK)])

        fire(0, *bufs[0])

        def do_pair(k, carry):
            g0 = 2 * k

            @pl.when(g0 + 1 < nchunk)
            def _():
                fire(g0 + 1, *bufs[1])
            drain_out(g0, *bufs[0])

            @pl.when(g0 + 2 < nchunk)
            def _():
                fire(g0 + 2, *bufs[0])

            @pl.when(g0 + 1 < nchunk)
            def _():
                drain_out(g0 + 1, *bufs[1])
            return carry
        lax.fori_loop(0, (nchunk + 1) // 2, do_pair, 0)

    return gather_k


_gather_a = _make_gather(B * TA)
_gather_b = _make_gather(B * TB)


def _sigmoid(x):
    return 0.5 * jnp.tanh(0.5 * x) + 0.5


def _lstm_steps(x_ref, wih, whh, b, h, c, nsteps):
    def step(t, carry):
        h, c = carry
        xt = x_ref[t].astype(jnp.bfloat16)
        gates = jnp.dot(xt, wih, preferred_element_type=jnp.float32)
        gates = gates + jnp.dot(h.astype(jnp.bfloat16), whh,
                                preferred_element_type=jnp.float32)
        gates = gates + b
        i = _sigmoid(gates[:, :HID])
        f = _sigmoid(gates[:, HID:2 * HID])
        g = jnp.tanh(gates[:, 2 * HID:3 * HID])
        o = _sigmoid(gates[:, 3 * HID:])
        c = f * c + i * g
        h = o * jnp.tanh(c)
        return (h, c)
    return lax.fori_loop(0, nsteps, step, (h, c))


def _lstm_first(x_ref, wih_ref, whh_ref, bih_ref, bhh_ref, h_out, c_out):
    wih = wih_ref[...].astype(jnp.bfloat16)
    whh = whh_ref[...].astype(jnp.bfloat16)
    b = bih_ref[...] + bhh_ref[...]
    h0 = jnp.zeros((B, HID), jnp.float32)
    c0 = jnp.zeros((B, HID), jnp.float32)
    h, c = _lstm_steps(x_ref, wih, whh, b, h0, c0, TA)
    h_out[...] = h
    c_out[...] = c


def _lstm_second(x_ref, h_ref, c_ref, wih_ref, whh_ref, bih_ref, bhh_ref,
                 wcls_ref, bcls_ref, out_ref):
    wih = wih_ref[...].astype(jnp.bfloat16)
    whh = whh_ref[...].astype(jnp.bfloat16)
    b = bih_ref[...] + bhh_ref[...]
    h, c = _lstm_steps(x_ref, wih, whh, b, h_ref[...], c_ref[...], TB)
    out_ref[...] = (jnp.dot(h, wcls_ref[...], preferred_element_type=jnp.float32)
                    + bcls_ref[...])


def kernel(batch_input_ids, emb, W_ih, W_hh, b_ih, b_hh, W_cls, b_cls):
    idx_tb = batch_input_ids.T
    idx_a = idx_tb[:TA].reshape(32, B * TA // 32)
    idx_b = idx_tb[TA:].reshape(32, B * TB // 32)
    xa = _gather_a(emb, idx_a).reshape(TA, B, EMB)
    xb = _gather_b(emb, idx_b).reshape(TB, B, EMB)

    nlbl = W_cls.shape[0]
    wcls_pad = jnp.zeros((HID, 128), jnp.float32).at[:, :nlbl].set(W_cls.T)
    bcls_pad = jnp.zeros((1, 128), jnp.float32).at[0, :nlbl].set(b_cls)
    wih_t = W_ih.T
    whh_t = W_hh.T
    bih = b_ih.reshape(1, -1)
    bhh = b_hh.reshape(1, -1)

    h1, c1 = pl.pallas_call(
        _lstm_first,
        out_shape=(jax.ShapeDtypeStruct((B, HID), jnp.float32),
                   jax.ShapeDtypeStruct((B, HID), jnp.float32)),
    )(xa, wih_t, whh_t, bih, bhh)

    out = pl.pallas_call(
        _lstm_second,
        out_shape=jax.ShapeDtypeStruct((B, 128), jnp.float32),
    )(xb, h1, c1, wih_t, whh_t, bih, bhh, wcls_pad, bcls_pad)
    return out[:, :nlbl]


# speedup vs baseline: 1.0174x; 1.0174x over previous
import functools

import jax
import jax.numpy as jnp
from jax import lax
from jax.experimental import pallas as pl
from jax.experimental.pallas import tpu as pltpu
from jax.experimental.pallas import tpu_sc as plsc

EMB = 64
HID = 128
B = 1024
T = 50
TA = 25
TB = T - TA
CHUNK = 80
LANES = 16


def _make_gather(ntok):
    info = plsc.get_sparse_core_info()
    nc, ns = info.num_cores, info.num_subcores
    nw = nc * ns
    tok_w = ntok // nw
    nchunk = tok_w // CHUNK

    mesh = plsc.VectorSubcoreMesh(core_axis_name="c", subcore_axis_name="s")

    @functools.partial(
        pl.kernel,
        mesh=mesh,
        compiler_params=pltpu.CompilerParams(needs_layout_passes=False),
        out_type=jax.ShapeDtypeStruct((ntok, EMB), jnp.float32),
        scratch_types=[
            pltpu.VMEM((tok_w,), jnp.int32),
            pltpu.VMEM((CHUNK, EMB), jnp.float32),
            pltpu.VMEM((CHUNK, EMB), jnp.float32),
            pltpu.SemaphoreType.DMA,
            pltpu.SemaphoreType.DMA,
        ],
    )
    def gather_k(table_hbm, idx_hbm, out_hbm, idx_v, rows0, rows1,
                 sem0, sem1):
        wid = lax.axis_index("s") * nc + lax.axis_index("c")
        base = wid * tok_w
        pltpu.sync_copy(idx_hbm.at[wid], idx_v)
        bufs = ((rows0, sem0), (rows1, sem1))

        def fire(g, rows_v, sem):
            def fq(q, c):
                iv = idx_v[pl.ds(g * CHUNK + q * LANES, LANES)]
                for jj in range(LANES):
                    pltpu.async_copy(table_hbm.at[iv[jj]],
                                     rows_v.at[q * LANES + jj], sem)
                return c
            lax.fori_loop(0, CHUNK // LANES, fq, 0)

        def drain_out(g, rows_v, sem):
            def dj(j, c):
                pltpu.make_async_copy(table_hbm.at[0], rows_v.at[j],
                                      sem).wait()
                return c
            lax.fori_loop(0, CHUNK, dj, 0)
            pltpu.sync_copy(rows_v,
                            out_hbm.at[pl.ds(base + g * CHUNK, CHUNK)])

        fire(0, *bufs[0])

        def do_pair(k, carry):
            g0 = 2 * k

            @pl.when(g0 + 1 < nchunk)
            def _():
                fire(g0 + 1, *bufs[1])
            drain_out(g0, *bufs[0])

            @pl.when(g0 + 2 < nchunk)
            def _():
                fire(g0 + 2, *bufs[0])

            @pl.when(g0 + 1 < nchunk)
            def _():
                drain_out(g0 + 1, *bufs[1])
            return carry
        lax.fori_loop(0, (nchunk + 1) // 2, do_pair, 0)

    return gather_k


_gather_a = _make_gather(B * TA)
_gather_b = _make_gather(B * TB)


def _sigmoid(x):
    return 0.5 * jnp.tanh(0.5 * x) + 0.5


def _lstm_steps(x_ref, wih, whh, b, h, c, nsteps):
    def step(t, carry):
        h, c = carry
        xt = x_ref[t].astype(jnp.bfloat16)
        gates = jnp.dot(xt, wih, preferred_element_type=jnp.float32)
        gates = gates + jnp.dot(h.astype(jnp.bfloat16), whh,
                                preferred_element_type=jnp.float32)
        gates = gates + b
        i = _sigmoid(gates[:, :HID])
        f = _sigmoid(gates[:, HID:2 * HID])
        g = jnp.tanh(gates[:, 2 * HID:3 * HID])
        o = _sigmoid(gates[:, 3 * HID:])
        c = f * c + i * g
        h = o * jnp.tanh(c)
        return (h, c)
    return lax.fori_loop(0, nsteps, step, (h, c))


def _lstm_first(x_ref, wih_ref, whh_ref, bih_ref, bhh_ref, h_out, c_out):
    wih = wih_ref[...].astype(jnp.bfloat16)
    whh = whh_ref[...].astype(jnp.bfloat16)
    b = bih_ref[...] + bhh_ref[...]
    h0 = jnp.zeros((B, HID), jnp.float32)
    c0 = jnp.zeros((B, HID), jnp.float32)
    h, c = _lstm_steps(x_ref, wih, whh, b, h0, c0, TA)
    h_out[...] = h
    c_out[...] = c


def _lstm_second(x_ref, h_ref, c_ref, wih_ref, whh_ref, bih_ref, bhh_ref,
                 wcls_ref, bcls_ref, out_ref):
    wih = wih_ref[...].astype(jnp.bfloat16)
    whh = whh_ref[...].astype(jnp.bfloat16)
    b = bih_ref[...] + bhh_ref[...]
    h, c = _lstm_steps(x_ref, wih, whh, b, h_ref[...], c_ref[...], TB)
    out_ref[...] = (jnp.dot(h, wcls_ref[...], preferred_element_type=jnp.float32)
                    + bcls_ref[...])


def kernel(batch_input_ids, emb, W_ih, W_hh, b_ih, b_hh, W_cls, b_cls):
    idx_tb = batch_input_ids.T
    idx_a = idx_tb[:TA].reshape(32, B * TA // 32)
    idx_b = idx_tb[TA:].reshape(32, B * TB // 32)
    xa = _gather_a(emb, idx_a).reshape(TA, B, EMB)
    xb = _gather_b(emb, idx_b).reshape(TB, B, EMB)

    nlbl = W_cls.shape[0]
    wcls_pad = jnp.zeros((HID, 128), jnp.float32).at[:, :nlbl].set(W_cls.T)
    bcls_pad = jnp.zeros((1, 128), jnp.float32).at[0, :nlbl].set(b_cls)
    wih_t = W_ih.T
    whh_t = W_hh.T
    bih = b_ih.reshape(1, -1)
    bhh = b_hh.reshape(1, -1)

    h1, c1 = pl.pallas_call(
        _lstm_first,
        out_shape=(jax.ShapeDtypeStruct((B, HID), jnp.float32),
                   jax.ShapeDtypeStruct((B, HID), jnp.float32)),
    )(xa, wih_t, whh_t, bih, bhh)

    out = pl.pallas_call(
        _lstm_second,
        out_shape=jax.ShapeDtypeStruct((B, 128), jnp.float32),
    )(xb, h1, c1, wih_t, whh_t, bih, bhh, wcls_pad, bcls_pad)
    return out[:, :nlbl]
